# baseline (speedup 1.0000x reference)
import functools

import jax
import jax.numpy as jnp
from jax import lax
from jax.experimental import pallas as pl
from jax.experimental.pallas import tpu as pltpu


def _attn_kernel(gamma_ref,
                 x_ref,
                 w_ref,
                 b_ref,
                 o_ref,
                 *, g, cqp):
    gamma = gamma_ref[0, 0]
    for i in range(g):
        x = x_ref[i]
        xb = x.astype(jnp.bfloat16)

        proj = jnp.dot(w_ref[...], xb,
                       preferred_element_type=jnp.float32)
        proj = proj + b_ref[...]

        q = proj[:cqp].astype(jnp.bfloat16)
        k = proj[cqp:2 * cqp].astype(jnp.bfloat16)
        v = proj[2 * cqp:].astype(jnp.bfloat16)

        scores = lax.dot_general(q, k, (((0,), (0,)), ((), ())),
                                 preferred_element_type=jnp.float32)

        m = jnp.max(scores, axis=-1, keepdims=True)
        e = jnp.exp(scores - m)
        attn = e * pl.reciprocal(jnp.sum(e, axis=-1, keepdims=True),
                                 approx=True)

        out = lax.dot_general(v, attn.astype(jnp.bfloat16),
                              (((1,), (1,)), ((), ())),
                              preferred_element_type=jnp.float32)

        o_ref[i] = gamma * out + x


def _round_up(n, m):
    return -(-n // m) * m


def _pad_rows(a, rows):
    if rows == a.shape[0]:
        return a
    pad = jnp.zeros((rows - a.shape[0],) + a.shape[1:], a.dtype)
    return jnp.concatenate([a, pad], axis=0)


def kernel(x, wq, bq, wk, bk, wv, bv, gamma):
    B, C, W = x.shape
    Cq = wq.shape[0]
    Cqp = max(16, _round_up(Cq, 16))
    Cp = 2 * Cqp + C

    wqkv = jnp.concatenate(
        [_pad_rows(wq, Cqp), _pad_rows(wk, Cqp), wv], axis=0
    ).astype(jnp.bfloat16)
    bq2, bk2, bv2 = (jnp.reshape(b, (-1, 1)) for b in (bq, bk, bv))
    bqkv = jnp.concatenate(
        [_pad_rows(bq2, Cqp), _pad_rows(bk2, Cqp), bv2], axis=0
    ).astype(jnp.bfloat16).astype(jnp.float32)

    gamma_smem = jnp.asarray(gamma, jnp.float32).reshape(1, 1)

    G = 8
    grid = (B // G,)
    kernel_fn = functools.partial(_attn_kernel, g=G, cqp=Cqp)

    out = pl.pallas_call(
        kernel_fn,
        out_shape=jax.ShapeDtypeStruct((B, C, W), x.dtype),
        grid_spec=pltpu.PrefetchScalarGridSpec(
            num_scalar_prefetch=0,
            grid=grid,
            in_specs=[
                pl.BlockSpec(memory_space=pltpu.MemorySpace.SMEM),
                pl.BlockSpec((G, C, W), lambda b: (b, 0, 0)),
                pl.BlockSpec((Cp, C), lambda b: (0, 0)),
                pl.BlockSpec((Cp, 1), lambda b: (0, 0)),
            ],
            out_specs=pl.BlockSpec((G, C, W), lambda b: (b, 0, 0)),
        ),
        compiler_params=pltpu.CompilerParams(
            dimension_semantics=("parallel",),
            vmem_limit_bytes=64 << 20,
        ),
    )(gamma_smem, x, wqkv, bqkv)

    return out


# speedup vs baseline: 1.0394x; 1.0394x over previous
import functools

import jax
import jax.numpy as jnp
from jax import lax
from jax.experimental import pallas as pl
from jax.experimental.pallas import tpu as pltpu


def _attn_kernel(gamma_ref,
                 x_ref,
                 w_ref,
                 b_ref,
                 o_ref,
                 *, g, cqp):
    gamma = gamma_ref[0, 0]
    for i in range(g):
        x = x_ref[i]
        xb = x.astype(jnp.bfloat16)

        proj = jnp.dot(w_ref[...], xb,
                       preferred_element_type=jnp.float32)
        proj = proj + b_ref[...]

        q = proj[:cqp].astype(jnp.bfloat16)
        k = proj[cqp:2 * cqp].astype(jnp.bfloat16)
        v = proj[2 * cqp:].astype(jnp.bfloat16)

        scores = lax.dot_general(q, k, (((0,), (0,)), ((), ())),
                                 preferred_element_type=jnp.float32)

        m = jnp.max(scores, axis=-1, keepdims=True)
        e = jnp.exp(scores - m)
        attn = e * pl.reciprocal(jnp.sum(e, axis=-1, keepdims=True),
                                 approx=True)

        out = lax.dot_general(v, attn.astype(jnp.bfloat16),
                              (((1,), (1,)), ((), ())),
                              preferred_element_type=jnp.float32)

        o_ref[i] = gamma * out + x


def _round_up(n, m):
    return -(-n // m) * m


def _pad_rows(a, rows):
    if rows == a.shape[0]:
        return a
    pad = jnp.zeros((rows - a.shape[0],) + a.shape[1:], a.dtype)
    return jnp.concatenate([a, pad], axis=0)


def kernel(x, wq, bq, wk, bk, wv, bv, gamma):
    B, C, W = x.shape
    Cq = wq.shape[0]
    Cqp = max(16, _round_up(Cq, 16))
    Cp = 2 * Cqp + C

    wqkv = jnp.concatenate(
        [_pad_rows(wq, Cqp), _pad_rows(wk, Cqp), wv], axis=0
    ).astype(jnp.bfloat16)
    bq2, bk2, bv2 = (jnp.reshape(b, (-1, 1)) for b in (bq, bk, bv))
    bqkv = jnp.concatenate(
        [_pad_rows(bq2, Cqp), _pad_rows(bk2, Cqp), bv2], axis=0
    ).astype(jnp.bfloat16).astype(jnp.float32)

    gamma_smem = jnp.asarray(gamma, jnp.float32).reshape(1, 1)

    G = 4
    grid = (B // G,)
    kernel_fn = functools.partial(_attn_kernel, g=G, cqp=Cqp)

    out = pl.pallas_call(
        kernel_fn,
        out_shape=jax.ShapeDtypeStruct((B, C, W), x.dtype),
        grid_spec=pltpu.PrefetchScalarGridSpec(
            num_scalar_prefetch=0,
            grid=grid,
            in_specs=[
                pl.BlockSpec(memory_space=pltpu.MemorySpace.SMEM),
                pl.BlockSpec((G, C, W), lambda b: (b, 0, 0)),
                pl.BlockSpec((Cp, C), lambda b: (0, 0)),
                pl.BlockSpec((Cp, 1), lambda b: (0, 0)),
            ],
            out_specs=pl.BlockSpec((G, C, W), lambda b: (b, 0, 0)),
        ),
        compiler_params=pltpu.CompilerParams(
            dimension_semantics=("parallel",),
            vmem_limit_bytes=64 << 20,
        ),
    )(gamma_smem, x, wqkv, bqkv)

    return out
